# baseline (speedup 1.0000x reference)
import functools

import jax
import jax.numpy as jnp
from jax import lax
from jax.experimental import pallas as pl
from jax.experimental.pallas import tpu as pltpu
from jax.experimental.pallas import tpu_sc as plsc

_B = 16384
_F = 64
_NC = 2
_NS = 16
_L = 16
_NW = _NC * _NS
_BPW = _B // _NW
_CHUNK = 128
_NCHUNK = _BPW // _CHUNK
_GROUPS = _BPW // _L


def _mf_body(user_emb, movie_emb, user_bias, movie_bias, uidx_hbm, midx_hbm,
             out_hbm, uidx_v, midx_v, ue_v, me_v, ub_s, mb_s, ubrow_v, mbrow_v,
             out_v, sem):
    wid = lax.axis_index("s") * _NC + lax.axis_index("c")

    pltpu.sync_copy(uidx_hbm.at[wid], uidx_v)
    pltpu.sync_copy(midx_hbm.at[wid], midx_v)

    copies = []
    for c in range(_NCHUNK):
        dst = pl.ds(c * _CHUNK, _CHUNK)
        copies.append(pltpu.async_copy(user_emb.at[uidx_v.at[c]], ue_v.at[dst], sem))
        copies.append(pltpu.async_copy(movie_emb.at[midx_v.at[c]], me_v.at[dst], sem))

    for c in range(_NCHUNK):
        for k in range(_CHUNK // _L):
            sl = pl.ds(k * _L, _L)
            ubrow_v[c, sl] = lax.shift_right_logical(uidx_v[c, sl], 4)
            mbrow_v[c, sl] = lax.shift_right_logical(midx_v[c, sl], 4)
    for c in range(_NCHUNK):
        dst = pl.ds(c * _CHUNK, _CHUNK)
        copies.append(pltpu.async_copy(user_bias.at[ubrow_v.at[c]], ub_s.at[dst], sem))
        copies.append(pltpu.async_copy(movie_bias.at[mbrow_v.at[c]], mb_s.at[dst], sem))

    for cp in copies:
        cp.wait()

    def group_body(g, carry):
        row = g * _L + lax.iota(jnp.int32, _L)
        acc = jnp.zeros((_L,), jnp.float32)
        for j in range(_F):
            col = jnp.full((_L,), j, jnp.int32)
            u = plsc.load_gather(ue_v, [row, col])
            m = plsc.load_gather(me_v, [row, col])
            acc = acc + u * m
        chunk = lax.shift_right_logical(row, 7)
        pos = lax.bitwise_and(row, _CHUNK - 1)
        uvals = plsc.load_gather(uidx_v, [chunk, pos])
        mvals = plsc.load_gather(midx_v, [chunk, pos])
        ub = plsc.load_gather(ub_s, [row, lax.bitwise_and(uvals, _L - 1)])
        mb = plsc.load_gather(mb_s, [row, lax.bitwise_and(mvals, _L - 1)])
        out_v[pl.ds(g * _L, _L)] = acc + ub + mb + 3.5
        return carry

    lax.fori_loop(0, _GROUPS, group_body, 0)

    pltpu.sync_copy(out_v, out_hbm.at[pl.ds(wid * _BPW, _BPW)])


_mf_kernel = functools.partial(
    pl.kernel,
    out_type=jax.ShapeDtypeStruct((_B,), jnp.float32),
    mesh=plsc.VectorSubcoreMesh(core_axis_name="c", subcore_axis_name="s"),
    compiler_params=pltpu.CompilerParams(
        needs_layout_passes=False, use_tc_tiling_on_sc=False),
    scratch_types=[
        pltpu.VMEM((_NCHUNK, _CHUNK), jnp.int32),
        pltpu.VMEM((_NCHUNK, _CHUNK), jnp.int32),
        pltpu.VMEM((_BPW, _F), jnp.float32),
        pltpu.VMEM((_BPW, _F), jnp.float32),
        pltpu.VMEM((_BPW, _L), jnp.float32),
        pltpu.VMEM((_BPW, _L), jnp.float32),
        pltpu.VMEM((_NCHUNK, _CHUNK), jnp.int32),
        pltpu.VMEM((_NCHUNK, _CHUNK), jnp.int32),
        pltpu.VMEM((_BPW,), jnp.float32),
        pltpu.SemaphoreType.DMA,
    ],
)(_mf_body)


@jax.jit
def kernel(user_emb, movie_emb, user_bias, movie_bias, user_indices, movie_indices):
    uidx = user_indices.astype(jnp.int32).reshape(_NW, _NCHUNK, _CHUNK)
    midx = movie_indices.astype(jnp.int32).reshape(_NW, _NCHUNK, _CHUNK)
    ub = user_bias.reshape(user_bias.shape[0] // _L, _L)
    mb = movie_bias.reshape(movie_bias.shape[0] // _L, _L)
    return _mf_kernel(user_emb, movie_emb, ub, mb, uidx, midx)
